# baseline (speedup 1.0000x reference)
import functools

import jax
import jax.numpy as jnp
from jax import lax
from jax.experimental import pallas as pl
from jax.experimental.pallas import tpu as pltpu
from jax.experimental.pallas import tpu_sc as plsc

N = 10000
E = 320000
D = 128

NC = 2
NS = 16
NW = NC * NS
CHUNK = 64
NCHUNK = 160
EPW = CHUNK * NCHUNK
EP = NW * EPW
NPAD = 10240
RPS = NPAD // NS


def _sc_aggregate(feat, src_flat, dst_flat, zeros):
    mesh = plsc.VectorSubcoreMesh(core_axis_name="c", subcore_axis_name="s")

    @functools.partial(
        pl.kernel,
        out_type=jax.ShapeDtypeStruct((NC * NPAD, D), jnp.float32),
        mesh=mesh,
        scratch_types=[
            pltpu.VMEM((CHUNK,), jnp.int32),
            pltpu.VMEM((CHUNK,), jnp.int32),
            pltpu.VMEM((CHUNK, D), jnp.float32),
            pltpu.VMEM((CHUNK, D), jnp.float32),
            pltpu.VMEM_SHARED((NPAD, D), jnp.float32),
            pltpu.SemaphoreType.DMA,
            pltpu.SemaphoreType.DMA,
        ],
    )
    def agg_kernel(feat_hbm, src_hbm, dst_flat_hbm, zeros_hbm, out_hbm,
                   sidx_a, sidx_b, rows_a, rows_b, acc, sem_a, sem_b):
        cid = lax.axis_index("c")
        sid = lax.axis_index("s")
        wid = sid * NC + cid
        base = wid * EPW

        pltpu.sync_copy(zeros_hbm, acc.at[pl.ds(sid * RPS, RPS)])
        plsc.subcore_barrier()

        @pl.loop(0, NCHUNK)
        def _(j):
            pltpu.sync_copy(src_hbm.at[pl.ds(base + j * CHUNK, CHUNK)],
                            sidx_a)
            pltpu.sync_copy(dst_flat_hbm.at[pl.ds(base + j * CHUNK, CHUNK)],
                            sidx_b)
            pltpu.async_copy(feat_hbm.at[sidx_a], rows_a, sem_a).wait()
            pltpu.sync_copy(rows_a, acc.at[sidx_b], add=True)

        plsc.subcore_barrier()
        pltpu.sync_copy(
            acc.at[pl.ds(sid * RPS, RPS)],
            out_hbm.at[pl.ds(cid * NPAD + sid * RPS, RPS)],
        )

    return agg_kernel(feat, src_flat, dst_flat, zeros)


def _dense_layer(x, p0, p1, indeg, w_top, w_bot, b, gamma, beta, ln_relu):
    R = 2000

    def body(x_ref, p0_ref, p1_ref, d_ref, wt_ref, wb_ref, b_ref, g_ref,
             be_ref, o_ref):
        agg = (p0_ref[...] + p1_ref[...]) / d_ref[...]
        h = (
            jnp.dot(x_ref[...], wt_ref[...], preferred_element_type=jnp.float32)
            + jnp.dot(agg, wb_ref[...], preferred_element_type=jnp.float32)
            + b_ref[...]
        )
        if ln_relu:
            mu = jnp.mean(h, axis=-1, keepdims=True)
            var = jnp.mean((h - mu) ** 2, axis=-1, keepdims=True)
            h = (h - mu) * lax.rsqrt(var + 1e-5) * g_ref[...] + be_ref[...]
            h = jnp.maximum(h, 0.0)
        o_ref[...] = h

    row_spec = pl.BlockSpec((R, D), lambda i: (i, 0))
    full_spec = pl.BlockSpec((D, D), lambda i: (0, 0))
    vec_spec = pl.BlockSpec((1, D), lambda i: (0, 0))
    return pl.pallas_call(
        body,
        grid=(N // R,),
        in_specs=[
            row_spec, row_spec, row_spec,
            pl.BlockSpec((R, 1), lambda i: (i, 0)),
            full_spec, full_spec, vec_spec, vec_spec, vec_spec,
        ],
        out_specs=row_spec,
        out_shape=jax.ShapeDtypeStruct((N, D), jnp.float32),
    )(x, p0, p1, indeg, w_top, w_bot, b, gamma, beta)


def kernel(feat, g, in_deg, W1, b1, W2, b2, gamma, beta):
    zeros = jnp.zeros((RPS, D), jnp.float32)
    indeg = in_deg[:, None]
    b1r = b1[None, :]
    b2r = b2[None, :]
    gr = gamma[None, :]
    ber = beta[None, :]

    pad = EP - E
    src_flat = jnp.concatenate([g[0], jnp.zeros((pad,), jnp.int32)])
    dst_flat = jnp.concatenate([g[1], jnp.full((pad,), N, jnp.int32)])
    p = _sc_aggregate(feat, src_flat, dst_flat, zeros)
    h1 = _dense_layer(feat, p[:N], p[NPAD:NPAD + N], indeg, W1[:D], W1[D:],
                      b1r, gr, ber, True)
    p2 = _sc_aggregate(h1, src_flat, dst_flat, zeros)
    return _dense_layer(h1, p2[:N], p2[NPAD:NPAD + N], indeg, W2[:D], W2[D:],
                        b2r, gr, ber, False)


# speedup vs baseline: 1.8204x; 1.8204x over previous
import functools

import jax
import jax.numpy as jnp
from jax import lax
from jax.experimental import pallas as pl
from jax.experimental.pallas import tpu as pltpu
from jax.experimental.pallas import tpu_sc as plsc

N = 10000
E = 320000
D = 128

NC = 2
NS = 16
NW = NC * NS
CHUNK = 64
NCHUNK = 160
EPW = CHUNK * NCHUNK
EP = NW * EPW
NPAD = 10240
RPS = NPAD // NS


def _sc_aggregate(feat, src_flat, dst_flat, zeros):
    mesh = plsc.VectorSubcoreMesh(core_axis_name="c", subcore_axis_name="s")

    @functools.partial(
        pl.kernel,
        out_type=jax.ShapeDtypeStruct((NC * NPAD, D), jnp.float32),
        mesh=mesh,
        scratch_types=[
            pltpu.VMEM((CHUNK,), jnp.int32),
            pltpu.VMEM((CHUNK,), jnp.int32),
            pltpu.VMEM((CHUNK, D), jnp.float32),
            pltpu.VMEM((CHUNK, D), jnp.float32),
            pltpu.VMEM_SHARED((NPAD, D), jnp.float32),
            pltpu.SemaphoreType.DMA,
            pltpu.SemaphoreType.DMA,
        ],
    )
    def agg_kernel(feat_hbm, src_hbm, dst_flat_hbm, zeros_hbm, out_hbm,
                   sidx_a, sidx_b, rows_a, rows_b, acc, sem_a, sem_b):
        cid = lax.axis_index("c")
        sid = lax.axis_index("s")
        wid = sid * NC + cid
        base = wid * EPW

        pltpu.sync_copy(zeros_hbm, acc.at[pl.ds(sid * RPS, RPS)])
        plsc.subcore_barrier()

        @pl.loop(0, NCHUNK)
        def _(j):
            pltpu.sync_copy(src_hbm.at[pl.ds(base + j * CHUNK, CHUNK)],
                            sidx_a)
            pltpu.sync_copy(dst_flat_hbm.at[pl.ds(base + j * CHUNK, CHUNK)],
                            sidx_b)
            pltpu.async_copy(feat_hbm.at[sidx_a], rows_a, sem_a).wait()
            pltpu.sync_copy(rows_a, acc.at[sidx_b], add=True)

        plsc.subcore_barrier()
        pltpu.sync_copy(
            acc.at[pl.ds(sid * RPS, RPS)],
            out_hbm.at[pl.ds(cid * NPAD + sid * RPS, RPS)],
        )

    return agg_kernel(feat, src_flat, dst_flat, zeros)


def _dense_layer(x, p0, p1, indeg, w_top, w_bot, b, gamma, beta, ln_relu):
    R = 2000

    def body(x_ref, p0_ref, p1_ref, d_ref, wt_ref, wb_ref, b_ref, g_ref,
             be_ref, o_ref):
        agg = (p0_ref[...] + p1_ref[...]) / d_ref[...]
        h = (
            jnp.dot(x_ref[...], wt_ref[...], preferred_element_type=jnp.float32)
            + jnp.dot(agg, wb_ref[...], preferred_element_type=jnp.float32)
            + b_ref[...]
        )
        if ln_relu:
            mu = jnp.mean(h, axis=-1, keepdims=True)
            var = jnp.mean((h - mu) ** 2, axis=-1, keepdims=True)
            h = (h - mu) * lax.rsqrt(var + 1e-5) * g_ref[...] + be_ref[...]
            h = jnp.maximum(h, 0.0)
        o_ref[...] = h

    row_spec = pl.BlockSpec((R, D), lambda i: (i, 0))
    full_spec = pl.BlockSpec((D, D), lambda i: (0, 0))
    vec_spec = pl.BlockSpec((1, D), lambda i: (0, 0))
    return pl.pallas_call(
        body,
        grid=(N // R,),
        in_specs=[
            row_spec, row_spec, row_spec,
            pl.BlockSpec((R, 1), lambda i: (i, 0)),
            full_spec, full_spec, vec_spec, vec_spec, vec_spec,
        ],
        out_specs=row_spec,
        out_shape=jax.ShapeDtypeStruct((N, D), jnp.float32),
    )(x, p0, p1, indeg, w_top, w_bot, b, gamma, beta)


def kernel(feat, g, in_deg, W1, b1, W2, b2, gamma, beta):
    zeros = jnp.zeros((RPS, D), jnp.float32)
    indeg = in_deg[:, None]
    b1r = b1[None, :]
    b2r = b2[None, :]
    gr = gamma[None, :]
    ber = beta[None, :]

    pad = EP - E
    pad_dst = N + jnp.arange(pad, dtype=jnp.int32) % (NPAD - N)
    pad_src = jnp.arange(pad, dtype=jnp.int32) % N
    src_flat = jnp.concatenate([g[0], pad_src])
    dst_flat = jnp.concatenate([g[1], pad_dst])
    p = _sc_aggregate(feat, src_flat, dst_flat, zeros)
    h1 = _dense_layer(feat, p[:N], p[NPAD:NPAD + N], indeg, W1[:D], W1[D:],
                      b1r, gr, ber, True)
    p2 = _sc_aggregate(h1, src_flat, dst_flat, zeros)
    return _dense_layer(h1, p2[:N], p2[NPAD:NPAD + N], indeg, W2[:D], W2[D:],
                        b2r, gr, ber, False)


# speedup vs baseline: 3.9718x; 2.1818x over previous
import functools

import jax
import jax.numpy as jnp
from jax import lax
from jax.experimental import pallas as pl
from jax.experimental.pallas import tpu as pltpu
from jax.experimental.pallas import tpu_sc as plsc

N = 10000
E = 320000
D = 128

NC = 2
NS = 16
NW = NC * NS
CHUNK = 128
NCHUNK = 80
EPW = CHUNK * NCHUNK
EP = NW * EPW
NPAD = 10240
RPS = NPAD // NS


def _sc_aggregate(feat, src_flat, dst3, zeros):
    mesh = plsc.VectorSubcoreMesh(core_axis_name="c", subcore_axis_name="s")

    @functools.partial(
        pl.kernel,
        out_type=jax.ShapeDtypeStruct((NC * NPAD, D), jnp.float32),
        mesh=mesh,
        scratch_types=[
            pltpu.VMEM((CHUNK,), jnp.int32),
            pltpu.VMEM((CHUNK,), jnp.int32),
            pltpu.VMEM((NCHUNK, CHUNK), jnp.int32),
            pltpu.VMEM((CHUNK, D), jnp.float32),
            pltpu.VMEM((CHUNK, D), jnp.float32),
            pltpu.VMEM_SHARED((NPAD, D), jnp.float32),
            pltpu.SemaphoreType.DMA,
            pltpu.SemaphoreType.DMA,
            pltpu.SemaphoreType.DMA,
            pltpu.SemaphoreType.DMA,
            pltpu.SemaphoreType.DMA,
            pltpu.SemaphoreType.DMA,
        ],
    )
    def agg_kernel(feat_hbm, src_hbm, dst_hbm, zeros_hbm, out_hbm,
                   sidx_a, sidx_b, didx, rows_a, rows_b, acc,
                   sem_ga, sem_gb, sem_sa, sem_sb, sem_ia, sem_ib):
        cid = lax.axis_index("c")
        sid = lax.axis_index("s")
        wid = sid * NC + cid
        base = wid * EPW

        pltpu.sync_copy(dst_hbm.at[wid], didx)
        pltpu.sync_copy(zeros_hbm, acc.at[pl.ds(sid * RPS, RPS)])
        pltpu.sync_copy(src_hbm.at[pl.ds(base, CHUNK)], sidx_a)
        pltpu.sync_copy(src_hbm.at[pl.ds(base + CHUNK, CHUNK)], sidx_b)
        pltpu.async_copy(feat_hbm.at[sidx_a], rows_a, sem_ga)
        pltpu.async_copy(feat_hbm.at[sidx_b], rows_b, sem_gb)
        plsc.subcore_barrier()

        @pl.loop(0, NCHUNK, step=2)
        def _(j):
            nxt_a = base + lax.rem(j + 2, NCHUNK) * CHUNK
            nxt_b = base + lax.rem(j + 3, NCHUNK) * CHUNK

            pltpu.make_async_copy(feat_hbm.at[sidx_a], rows_a, sem_ga).wait()
            pltpu.async_copy(rows_a, acc.at[didx.at[j]], sem_sa, add=True)
            pltpu.async_copy(src_hbm.at[pl.ds(nxt_a, CHUNK)], sidx_a, sem_ia)

            pltpu.make_async_copy(feat_hbm.at[sidx_b], rows_b, sem_gb).wait()
            pltpu.async_copy(rows_b, acc.at[didx.at[j + 1]], sem_sb, add=True)
            pltpu.async_copy(src_hbm.at[pl.ds(nxt_b, CHUNK)], sidx_b, sem_ib)

            pltpu.make_async_copy(rows_a, acc.at[didx.at[j]], sem_sa).wait()
            pltpu.make_async_copy(src_hbm.at[pl.ds(nxt_a, CHUNK)], sidx_a,
                                  sem_ia).wait()
            pltpu.async_copy(feat_hbm.at[sidx_a], rows_a, sem_ga)

            pltpu.make_async_copy(rows_b, acc.at[didx.at[j + 1]],
                                  sem_sb).wait()
            pltpu.make_async_copy(src_hbm.at[pl.ds(nxt_b, CHUNK)], sidx_b,
                                  sem_ib).wait()
            pltpu.async_copy(feat_hbm.at[sidx_b], rows_b, sem_gb)

        pltpu.make_async_copy(feat_hbm.at[sidx_a], rows_a, sem_ga).wait()
        pltpu.make_async_copy(feat_hbm.at[sidx_b], rows_b, sem_gb).wait()
        plsc.subcore_barrier()
        pltpu.sync_copy(
            acc.at[pl.ds(sid * RPS, RPS)],
            out_hbm.at[pl.ds(cid * NPAD + sid * RPS, RPS)],
        )

    return agg_kernel(feat, src_flat, dst3, zeros)


def _dense_layer(x, p0, p1, indeg, w_top, w_bot, b, gamma, beta, ln_relu):
    R = 2000

    def body(x_ref, p0_ref, p1_ref, d_ref, wt_ref, wb_ref, b_ref, g_ref,
             be_ref, o_ref):
        agg = (p0_ref[...] + p1_ref[...]) / d_ref[...]
        h = (
            jnp.dot(x_ref[...], wt_ref[...], preferred_element_type=jnp.float32)
            + jnp.dot(agg, wb_ref[...], preferred_element_type=jnp.float32)
            + b_ref[...]
        )
        if ln_relu:
            mu = jnp.mean(h, axis=-1, keepdims=True)
            var = jnp.mean((h - mu) ** 2, axis=-1, keepdims=True)
            h = (h - mu) * lax.rsqrt(var + 1e-5) * g_ref[...] + be_ref[...]
            h = jnp.maximum(h, 0.0)
        o_ref[...] = h

    row_spec = pl.BlockSpec((R, D), lambda i: (i, 0))
    full_spec = pl.BlockSpec((D, D), lambda i: (0, 0))
    vec_spec = pl.BlockSpec((1, D), lambda i: (0, 0))
    return pl.pallas_call(
        body,
        grid=(N // R,),
        in_specs=[
            row_spec, row_spec, row_spec,
            pl.BlockSpec((R, 1), lambda i: (i, 0)),
            full_spec, full_spec, vec_spec, vec_spec, vec_spec,
        ],
        out_specs=row_spec,
        out_shape=jax.ShapeDtypeStruct((N, D), jnp.float32),
    )(x, p0, p1, indeg, w_top, w_bot, b, gamma, beta)


def kernel(feat, g, in_deg, W1, b1, W2, b2, gamma, beta):
    zeros = jnp.zeros((RPS, D), jnp.float32)
    indeg = in_deg[:, None]
    b1r = b1[None, :]
    b2r = b2[None, :]
    gr = gamma[None, :]
    ber = beta[None, :]

    pad = EP - E
    pad_dst = N + jnp.arange(pad, dtype=jnp.int32) % (NPAD - N)
    pad_src = jnp.arange(pad, dtype=jnp.int32) % N
    src_flat = jnp.concatenate([g[0], pad_src])
    dst3 = jnp.concatenate([g[1], pad_dst]).reshape(NW, NCHUNK, CHUNK)
    p = _sc_aggregate(feat, src_flat, dst3, zeros)
    h1 = _dense_layer(feat, p[:N], p[NPAD:NPAD + N], indeg, W1[:D], W1[D:],
                      b1r, gr, ber, True)
    p2 = _sc_aggregate(h1, src_flat, dst3, zeros)
    return _dense_layer(h1, p2[:N], p2[NPAD:NPAD + N], indeg, W2[:D], W2[D:],
                        b2r, gr, ber, False)


# speedup vs baseline: 4.5477x; 1.1450x over previous
import functools

import jax
import jax.numpy as jnp
from jax import lax
from jax.experimental import pallas as pl
from jax.experimental.pallas import tpu as pltpu
from jax.experimental.pallas import tpu_sc as plsc

N = 10000
E = 320000
D = 128

NC = 2
NS = 16
NW = NC * NS
CHUNK = 128
NCHUNK = 81
EPW = CHUNK * NCHUNK
EP = NW * EPW
NPAD = 10112
RPS = NPAD // NS


def _sc_aggregate(feat, src_flat, dst_flat, zeros):
    mesh = plsc.VectorSubcoreMesh(core_axis_name="c", subcore_axis_name="s")

    @functools.partial(
        pl.kernel,
        out_type=jax.ShapeDtypeStruct((NC * NPAD, D), jnp.float32),
        mesh=mesh,
        scratch_types=[
            pltpu.VMEM((3, CHUNK), jnp.int32),
            pltpu.VMEM((3, CHUNK), jnp.int32),
            pltpu.VMEM((CHUNK, D), jnp.float32),
            pltpu.VMEM((CHUNK, D), jnp.float32),
            pltpu.VMEM((CHUNK, D), jnp.float32),
            pltpu.VMEM_SHARED((NPAD, D), jnp.float32),
            pltpu.SemaphoreType.DMA,
            pltpu.SemaphoreType.DMA,
            pltpu.SemaphoreType.DMA,
            pltpu.SemaphoreType.DMA,
            pltpu.SemaphoreType.DMA,
            pltpu.SemaphoreType.DMA,
            pltpu.SemaphoreType.DMA,
            pltpu.SemaphoreType.DMA,
            pltpu.SemaphoreType.DMA,
            pltpu.SemaphoreType.DMA,
            pltpu.SemaphoreType.DMA,
            pltpu.SemaphoreType.DMA,
        ],
    )
    def agg_kernel(feat_hbm, src_hbm, dst_hbm, zeros_hbm, out_hbm,
                   sidx, didx, rows0, rows1, rows2, acc,
                   g0, g1, g2, s0, s1, s2, i0, i1, i2, d0, d1, d2):
        cid = lax.axis_index("c")
        sid = lax.axis_index("s")
        wid = sid * NC + cid
        base = wid * EPW
        rows = (rows0, rows1, rows2)
        sem_g = (g0, g1, g2)
        sem_s = (s0, s1, s2)
        sem_i = (i0, i1, i2)
        sem_d = (d0, d1, d2)

        pltpu.sync_copy(zeros_hbm, acc.at[pl.ds(sid * RPS, RPS)])
        for x in range(3):
            pltpu.sync_copy(src_hbm.at[pl.ds(base + x * CHUNK, CHUNK)],
                            sidx.at[x])
            pltpu.async_copy(dst_hbm.at[pl.ds(base + x * CHUNK, CHUNK)],
                             didx.at[x], sem_d[x])
            pltpu.async_copy(feat_hbm.at[sidx.at[x]], rows[x], sem_g[x])
        plsc.subcore_barrier()

        @pl.loop(0, NCHUNK, step=3)
        def _(j):
            for x in range(3):
                nxt = base + lax.rem(j + 3 + x, NCHUNK) * CHUNK
                pltpu.make_async_copy(feat_hbm.at[sidx.at[x]], rows[x],
                                      sem_g[x]).wait()
                pltpu.make_async_copy(dst_hbm.at[pl.ds(nxt, CHUNK)],
                                      didx.at[x], sem_d[x]).wait()
                pltpu.async_copy(rows[x], acc.at[didx.at[x]], sem_s[x],
                                 add=True)
                pltpu.async_copy(src_hbm.at[pl.ds(nxt, CHUNK)], sidx.at[x],
                                 sem_i[x])

            for x in range(3):
                nxt = base + lax.rem(j + 3 + x, NCHUNK) * CHUNK
                pltpu.make_async_copy(rows[x], acc.at[didx.at[x]],
                                      sem_s[x]).wait()
                pltpu.async_copy(dst_hbm.at[pl.ds(nxt, CHUNK)], didx.at[x],
                                 sem_d[x])
                pltpu.make_async_copy(src_hbm.at[pl.ds(nxt, CHUNK)],
                                      sidx.at[x], sem_i[x]).wait()
                pltpu.async_copy(feat_hbm.at[sidx.at[x]], rows[x], sem_g[x])

        for x in range(3):
            pltpu.make_async_copy(feat_hbm.at[sidx.at[x]], rows[x],
                                  sem_g[x]).wait()
            pltpu.make_async_copy(dst_hbm.at[pl.ds(base, CHUNK)], didx.at[x],
                                  sem_d[x]).wait()
        plsc.subcore_barrier()
        pltpu.sync_copy(
            acc.at[pl.ds(sid * RPS, RPS)],
            out_hbm.at[pl.ds(cid * NPAD + sid * RPS, RPS)],
        )

    return agg_kernel(feat, src_flat, dst_flat, zeros)


def _dense_layer(x, p0, p1, indeg, w_top, w_bot, b, gamma, beta, ln_relu):
    R = 2000

    def body(x_ref, p0_ref, p1_ref, d_ref, wt_ref, wb_ref, b_ref, g_ref,
             be_ref, o_ref):
        agg = (p0_ref[...] + p1_ref[...]) / d_ref[...]
        h = (
            jnp.dot(x_ref[...], wt_ref[...], preferred_element_type=jnp.float32)
            + jnp.dot(agg, wb_ref[...], preferred_element_type=jnp.float32)
            + b_ref[...]
        )
        if ln_relu:
            mu = jnp.mean(h, axis=-1, keepdims=True)
            var = jnp.mean((h - mu) ** 2, axis=-1, keepdims=True)
            h = (h - mu) * lax.rsqrt(var + 1e-5) * g_ref[...] + be_ref[...]
            h = jnp.maximum(h, 0.0)
        o_ref[...] = h

    row_spec = pl.BlockSpec((R, D), lambda i: (i, 0))
    full_spec = pl.BlockSpec((D, D), lambda i: (0, 0))
    vec_spec = pl.BlockSpec((1, D), lambda i: (0, 0))
    return pl.pallas_call(
        body,
        grid=(N // R,),
        in_specs=[
            row_spec, row_spec, row_spec,
            pl.BlockSpec((R, 1), lambda i: (i, 0)),
            full_spec, full_spec, vec_spec, vec_spec, vec_spec,
        ],
        out_specs=row_spec,
        out_shape=jax.ShapeDtypeStruct((N, D), jnp.float32),
    )(x, p0, p1, indeg, w_top, w_bot, b, gamma, beta)


def kernel(feat, g, in_deg, W1, b1, W2, b2, gamma, beta):
    zeros = jnp.zeros((RPS, D), jnp.float32)
    indeg = in_deg[:, None]
    b1r = b1[None, :]
    b2r = b2[None, :]
    gr = gamma[None, :]
    ber = beta[None, :]

    pad = EP - E
    pad_dst = N + jnp.arange(pad, dtype=jnp.int32) % (NPAD - N)
    pad_src = jnp.arange(pad, dtype=jnp.int32) % N
    src_flat = jnp.concatenate([g[0], pad_src])
    dst_flat = jnp.concatenate([g[1], pad_dst])
    p = _sc_aggregate(feat, src_flat, dst_flat, zeros)
    h1 = _dense_layer(feat, p[:N], p[NPAD:NPAD + N], indeg, W1[:D], W1[D:],
                      b1r, gr, ber, True)
    p2 = _sc_aggregate(h1, src_flat, dst_flat, zeros)
    return _dense_layer(h1, p2[:N], p2[NPAD:NPAD + N], indeg, W2[:D], W2[D:],
                        b2r, gr, ber, False)


# speedup vs baseline: 4.7481x; 1.0441x over previous
import functools

import jax
import jax.numpy as jnp
from jax import lax
from jax.experimental import pallas as pl
from jax.experimental.pallas import tpu as pltpu
from jax.experimental.pallas import tpu_sc as plsc

N = 10000
E = 320000
D = 128

NC = 2
NS = 16
NW = NC * NS
CHUNK = 128
NCHUNK = 81
EPW = CHUNK * NCHUNK
EP = NW * EPW
NPAD = 10112
RPS = NPAD // NS


def _sc_aggregate(feat, src_flat, dst_flat, zeros):
    mesh = plsc.VectorSubcoreMesh(core_axis_name="c", subcore_axis_name="s")

    @functools.partial(
        pl.kernel,
        out_type=[jax.ShapeDtypeStruct((NPAD, D), jnp.float32),
                  jax.ShapeDtypeStruct((NPAD, D), jnp.float32)],
        mesh=mesh,
        scratch_types=[
            pltpu.VMEM((3, CHUNK), jnp.int32),
            pltpu.VMEM((3, CHUNK), jnp.int32),
            pltpu.VMEM((CHUNK, D), jnp.float32),
            pltpu.VMEM((CHUNK, D), jnp.float32),
            pltpu.VMEM((CHUNK, D), jnp.float32),
            pltpu.VMEM_SHARED((NPAD, D), jnp.float32),
            pltpu.SemaphoreType.DMA,
            pltpu.SemaphoreType.DMA,
            pltpu.SemaphoreType.DMA,
            pltpu.SemaphoreType.DMA,
            pltpu.SemaphoreType.DMA,
            pltpu.SemaphoreType.DMA,
            pltpu.SemaphoreType.DMA,
            pltpu.SemaphoreType.DMA,
            pltpu.SemaphoreType.DMA,
            pltpu.SemaphoreType.DMA,
            pltpu.SemaphoreType.DMA,
            pltpu.SemaphoreType.DMA,
        ],
    )
    def agg_kernel(feat_hbm, src_hbm, dst_hbm, zeros_hbm, out0_hbm, out1_hbm,
                   sidx, didx, rows0, rows1, rows2, acc,
                   g0, g1, g2, s0, s1, s2, i0, i1, i2, d0, d1, d2):
        cid = lax.axis_index("c")
        sid = lax.axis_index("s")
        wid = sid * NC + cid
        base = wid * EPW
        rows = (rows0, rows1, rows2)
        sem_g = (g0, g1, g2)
        sem_s = (s0, s1, s2)
        sem_i = (i0, i1, i2)
        sem_d = (d0, d1, d2)

        pltpu.sync_copy(zeros_hbm, acc.at[pl.ds(sid * RPS, RPS)])
        for x in range(3):
            pltpu.sync_copy(src_hbm.at[pl.ds(base + x * CHUNK, CHUNK)],
                            sidx.at[x])
            pltpu.async_copy(dst_hbm.at[pl.ds(base + x * CHUNK, CHUNK)],
                             didx.at[x], sem_d[x])
            pltpu.async_copy(feat_hbm.at[sidx.at[x]], rows[x], sem_g[x])
        plsc.subcore_barrier()

        @pl.loop(0, NCHUNK, step=3)
        def _(j):
            for x in range(3):
                nxt = base + lax.rem(j + 3 + x, NCHUNK) * CHUNK
                pltpu.make_async_copy(feat_hbm.at[sidx.at[x]], rows[x],
                                      sem_g[x]).wait()
                pltpu.make_async_copy(dst_hbm.at[pl.ds(nxt, CHUNK)],
                                      didx.at[x], sem_d[x]).wait()
                pltpu.async_copy(rows[x], acc.at[didx.at[x]], sem_s[x],
                                 add=True)
                pltpu.async_copy(src_hbm.at[pl.ds(nxt, CHUNK)], sidx.at[x],
                                 sem_i[x])

            for x in range(3):
                nxt = base + lax.rem(j + 3 + x, NCHUNK) * CHUNK
                pltpu.make_async_copy(rows[x], acc.at[didx.at[x]],
                                      sem_s[x]).wait()
                pltpu.async_copy(dst_hbm.at[pl.ds(nxt, CHUNK)], didx.at[x],
                                 sem_d[x])
                pltpu.make_async_copy(src_hbm.at[pl.ds(nxt, CHUNK)],
                                      sidx.at[x], sem_i[x]).wait()
                pltpu.async_copy(feat_hbm.at[sidx.at[x]], rows[x], sem_g[x])

        for x in range(3):
            pltpu.make_async_copy(feat_hbm.at[sidx.at[x]], rows[x],
                                  sem_g[x]).wait()
            pltpu.make_async_copy(dst_hbm.at[pl.ds(base, CHUNK)], didx.at[x],
                                  sem_d[x]).wait()
        plsc.subcore_barrier()

        @pl.when(cid == 0)
        def _():
            pltpu.sync_copy(acc.at[pl.ds(sid * RPS, RPS)],
                            out0_hbm.at[pl.ds(sid * RPS, RPS)])

        @pl.when(cid == 1)
        def _():
            pltpu.sync_copy(acc.at[pl.ds(sid * RPS, RPS)],
                            out1_hbm.at[pl.ds(sid * RPS, RPS)])

    return agg_kernel(feat, src_flat, dst_flat, zeros)


def _dense_layer(x, p0, p1, indeg, w_top, w_bot, b, gamma, beta, ln_relu):
    R = 2000

    def body(x_ref, p0_ref, p1_ref, d_ref, wt_ref, wb_ref, b_ref, g_ref,
             be_ref, o_ref):
        agg = (p0_ref[...] + p1_ref[...]) / d_ref[...]
        h = (
            jnp.dot(x_ref[...], wt_ref[...], preferred_element_type=jnp.float32)
            + jnp.dot(agg, wb_ref[...], preferred_element_type=jnp.float32)
            + b_ref[...]
        )
        if ln_relu:
            mu = jnp.mean(h, axis=-1, keepdims=True)
            var = jnp.mean((h - mu) ** 2, axis=-1, keepdims=True)
            h = (h - mu) * lax.rsqrt(var + 1e-5) * g_ref[...] + be_ref[...]
            h = jnp.maximum(h, 0.0)
        o_ref[...] = h

    row_spec = pl.BlockSpec((R, D), lambda i: (i, 0))
    full_spec = pl.BlockSpec((D, D), lambda i: (0, 0))
    vec_spec = pl.BlockSpec((1, D), lambda i: (0, 0))
    return pl.pallas_call(
        body,
        grid=(N // R,),
        in_specs=[
            row_spec, row_spec, row_spec,
            pl.BlockSpec((R, 1), lambda i: (i, 0)),
            full_spec, full_spec, vec_spec, vec_spec, vec_spec,
        ],
        out_specs=row_spec,
        out_shape=jax.ShapeDtypeStruct((N, D), jnp.float32),
    )(x, p0, p1, indeg, w_top, w_bot, b, gamma, beta)


def kernel(feat, g, in_deg, W1, b1, W2, b2, gamma, beta):
    zeros = jnp.zeros((RPS, D), jnp.float32)
    indeg = in_deg[:, None]
    b1r = b1[None, :]
    b2r = b2[None, :]
    gr = gamma[None, :]
    ber = beta[None, :]

    pad = EP - E
    pad_dst = N + jnp.arange(pad, dtype=jnp.int32) % (NPAD - N)
    pad_src = jnp.arange(pad, dtype=jnp.int32) % N
    src_flat = jnp.concatenate([g[0], pad_src])
    dst_flat = jnp.concatenate([g[1], pad_dst])
    p0, p1 = _sc_aggregate(feat, src_flat, dst_flat, zeros)
    h1 = _dense_layer(feat, p0, p1, indeg, W1[:D], W1[D:],
                      b1r, gr, ber, True)
    q0, q1 = _sc_aggregate(h1, src_flat, dst_flat, zeros)
    return _dense_layer(h1, q0, q1, indeg, W2[:D], W2[D:],
                        b2r, gr, ber, False)
